# baseline (speedup 1.0000x reference)
import functools

import jax
import jax.numpy as jnp
from jax import lax
from jax.experimental import pallas as pl
from jax.experimental.pallas import tpu as pltpu
from jax.experimental.pallas import tpu_sc as plsc

N = 10000
E = 320000
NFEAT = 128
NHID = 64
NHEADS = 4
ALPHA = 0.2
EPS = 1e-5

NC = 2
NS = 16
CHUNK = 80
ZROWS = 640
ZLAST = N - 15 * ZROWS
BN = 400


def _proj1_body(x_ref, wt_ref, b_ref, p_ref, hp_ref, src_ref):
    acc = jnp.dot(x_ref[...], wt_ref[...],
                  preferred_element_type=jnp.float32) + b_ref[...]
    hp_ref[0] = acc[:, :NFEAT]
    hp_ref[1] = acc[:, NFEAT:]
    src_ref[...] = jnp.dot(acc, p_ref[...], preferred_element_type=jnp.float32)


def _proj1(x, wt, bvec, p):
    return pl.pallas_call(
        _proj1_body,
        grid=(N // BN,),
        in_specs=[
            pl.BlockSpec((BN, NFEAT), lambda i: (i, 0)),
            pl.BlockSpec((NFEAT, 2 * NFEAT), lambda i: (0, 0)),
            pl.BlockSpec((1, 2 * NFEAT), lambda i: (0, 0)),
            pl.BlockSpec((2 * NFEAT, 8), lambda i: (0, 0)),
        ],
        out_specs=[
            pl.BlockSpec((2, BN, NFEAT), lambda i: (0, i, 0)),
            pl.BlockSpec((BN, 8), lambda i: (i, 0)),
        ],
        out_shape=[
            jax.ShapeDtypeStruct((2, N, NFEAT), jnp.float32),
            jax.ShapeDtypeStruct((N, 8), jnp.float32),
        ],
    )(x, wt, bvec, p)


def _ln(x, g, b):
    m = jnp.mean(x, axis=1, keepdims=True)
    v = jnp.mean((x - m) ** 2, axis=1, keepdims=True)
    return (x - m) / jnp.sqrt(v + EPS) * g + b


def _elu(x):
    return jnp.where(x > 0, x, jnp.exp(x) - 1.0)


def _mid_body(hp_ref, wot_ref, bo_ref, g_ref, be_ref, p2_ref, out_ref,
              src2_ref):
    parts = []
    for c in range(2):
        hcat = hp_ref[c]
        for k in range(2):
            h = 2 * c + k
            sl = hcat[:, k * NHID:(k + 1) * NHID]
            g = g_ref[0:1, h * NHID:(h + 1) * NHID]
            be = be_ref[0:1, h * NHID:(h + 1) * NHID]
            parts.append(_elu(_ln(sl, g, be)))
    x = jnp.concatenate(parts, axis=1)
    acc = jnp.dot(x, wot_ref[...], preferred_element_type=jnp.float32) + bo_ref[...]
    out_ref[...] = acc
    src2_ref[...] = jnp.dot(acc, p2_ref[...], preferred_element_type=jnp.float32)


def _mid(hp, wot, bo, g1v, be1v, p2):
    return pl.pallas_call(
        _mid_body,
        grid=(N // BN,),
        in_specs=[
            pl.BlockSpec((2, BN, NFEAT), lambda i: (0, i, 0)),
            pl.BlockSpec((2 * NFEAT, NFEAT), lambda i: (0, 0)),
            pl.BlockSpec((1, NFEAT), lambda i: (0, 0)),
            pl.BlockSpec((1, 2 * NFEAT), lambda i: (0, 0)),
            pl.BlockSpec((1, 2 * NFEAT), lambda i: (0, 0)),
            pl.BlockSpec((NFEAT, 2), lambda i: (0, 0)),
        ],
        out_specs=[
            pl.BlockSpec((BN, NFEAT), lambda i: (i, 0)),
            pl.BlockSpec((BN, 2), lambda i: (i, 0)),
        ],
        out_shape=[
            jax.ShapeDtypeStruct((N, NFEAT), jnp.float32),
            jax.ShapeDtypeStruct((N, 2), jnp.float32),
        ],
    )(hp, wot, bo, g1v, be1v, p2)


def _fin_body(p_ref, go_ref, beo_ref, out_ref):
    sm = p_ref[0] + p_ref[1]
    out_ref[...] = _elu(_ln(sm, go_ref[...], beo_ref[...]))


def _fin(part, gov, beov):
    return pl.pallas_call(
        _fin_body,
        grid=(N // BN,),
        in_specs=[
            pl.BlockSpec((2, BN, NFEAT), lambda i: (0, i, 0)),
            pl.BlockSpec((1, NFEAT), lambda i: (0, 0)),
            pl.BlockSpec((1, NFEAT), lambda i: (0, 0)),
        ],
        out_specs=pl.BlockSpec((BN, NFEAT), lambda i: (i, 0)),
        out_shape=jax.ShapeDtypeStruct((N, NFEAT), jnp.float32),
    )(part, gov, beov)


def _weights16(sa, sb):
    s = sa + sb
    return jnp.exp(-jnp.where(s >= 0, s, ALPHA * s))


def _scale_rows(rbuf, sl, w0, w1, nvec_lo, nvec_hi):
    @plsc.parallel_loop(0, CHUNK // 16, step=1, unroll=5)
    def _(g):
        gb = pl.multiple_of(g * 16, 16)
        wa_v = w0[pl.ds(gb, 16)]
        wb_v = w1[pl.ds(gb, 16)]
        for l in range(16):
            e = gb + l
            wa = wa_v[l]
            wb = wb_v[l]
            for j in range(nvec_lo):
                d = pl.ds(j * 16, 16)
                rbuf[sl, e, d] = rbuf[sl, e, d] * wa
            for j in range(nvec_lo, nvec_hi):
                d = pl.ds(j * 16, 16)
                rbuf[sl, e, d] = rbuf[sl, e, d] * wb


def _zero_acc(z_h, acc, s):
    @pl.when(s < NS - 1)
    def _():
        pltpu.sync_copy(z_h, acc.at[pl.ds(pl.multiple_of(s * ZROWS, 8), ZROWS)])

    @pl.when(s == NS - 1)
    def _():
        pltpu.sync_copy(z_h.at[pl.ds(0, ZLAST)],
                        acc.at[pl.ds((NS - 1) * ZROWS, ZLAST)])


def _drain_acc(acc, out_h, s, base):
    @pl.when(s < NS - 1)
    def _():
        pltpu.sync_copy(acc.at[pl.ds(pl.multiple_of(s * ZROWS, 8), ZROWS)],
                        out_h.at[pl.ds(pl.multiple_of(base + s * ZROWS, 8),
                                       ZROWS)])

    @pl.when(s == NS - 1)
    def _():
        pltpu.sync_copy(acc.at[pl.ds((NS - 1) * ZROWS, ZLAST)],
                        out_h.at[pl.ds(pl.multiple_of(base + (NS - 1) * ZROWS, 8),
                                       ZLAST)])


NCH1 = (E // NS) // CHUNK
NCH2 = (E // (NS * NC)) // CHUNK
NSLOT = 3


def _agg_body_common(layer1, c, s, row_h, col_h, tab_h, src_h, srcb_h, z_h,
                     out_h, rowi, coli, rowsc, gidx, i0, i2, s0, s1, s2, s3,
                     w0, w1, rbuf, acc, isems, gsems, ssems):
    if layer1:
        epw = E // NS
        nch = NCH1
        base = s * epw
        h0 = 2 * c
    else:
        epw = E // (NS * NC)
        nch = NCH2
        base = (s * NC + c) * epw

    def idx_load(q, sl):
        b = pl.multiple_of(base + q * CHUNK, 16)
        pltpu.async_copy(row_h.at[pl.ds(b, CHUNK)], rowi.at[sl], isems[sl])
        pltpu.async_copy(col_h.at[pl.ds(b, CHUNK)], coli.at[sl], isems[sl])

    def idx_wait(q, sl):
        b = pl.multiple_of(base + q * CHUNK, 16)
        pltpu.make_async_copy(row_h.at[pl.ds(b, CHUNK)], rowi.at[sl],
                              isems[sl]).wait()
        pltpu.make_async_copy(col_h.at[pl.ds(b, CHUNK)], coli.at[sl],
                              isems[sl]).wait()

    def gather_fire(sl):
        for k in range(CHUNK // 16):
            d = pl.ds(k * 16, 16)
            r = rowi[sl, d]
            cc = coli[sl, d]
            if layer1:
                gidx[sl, d] = cc + c * N
                i0[sl, d] = r * 8 + h0
                i2[sl, d] = cc * 8 + (h0 + 4)
            else:
                gidx[sl, d] = cc
                i0[sl, d] = r * 2
                i2[sl, d] = cc * 2 + 1
        pltpu.async_copy(tab_h.at[gidx.at[sl]], rbuf.at[sl], gsems[sl])
        pltpu.async_copy(src_h.at[i0.at[sl]], s0.at[sl], gsems[sl])
        pltpu.async_copy(src_h.at[i2.at[sl]], s2.at[sl], gsems[sl])
        if layer1:
            pltpu.async_copy(srcb_h.at[i0.at[sl]], s1.at[sl], gsems[sl])
            pltpu.async_copy(srcb_h.at[i2.at[sl]], s3.at[sl], gsems[sl])

    def gather_drain(sl):
        pltpu.make_async_copy(tab_h.at[gidx.at[sl]], rbuf.at[sl],
                              gsems[sl]).wait()
        pltpu.make_async_copy(src_h.at[i0.at[sl]], s0.at[sl], gsems[sl]).wait()
        pltpu.make_async_copy(src_h.at[i2.at[sl]], s2.at[sl], gsems[sl]).wait()
        if layer1:
            pltpu.make_async_copy(srcb_h.at[i0.at[sl]], s1.at[sl],
                                  gsems[sl]).wait()
            pltpu.make_async_copy(srcb_h.at[i2.at[sl]], s3.at[sl],
                                  gsems[sl]).wait()

    def scatter_wait(sl):
        pltpu.make_async_copy(rbuf.at[sl], acc.at[rowsc.at[sl]],
                              ssems[sl]).wait()

    def compute(sl):
        for k in range(CHUNK // 16):
            d = pl.ds(k * 16, 16)
            rowsc[sl, d] = rowi[sl, d]
            if layer1:
                w0[d] = _weights16(s0[sl, d], s2[sl, d])
                w1[d] = _weights16(s1[sl, d], s3[sl, d])
            else:
                w0[d] = _weights16(s0[sl, d], s2[sl, d])
        if layer1:
            _scale_rows(rbuf, sl, w0, w1, 4, 8)
        else:
            _scale_rows(rbuf, sl, w0, w0, 8, 8)

    _zero_acc(z_h, acc, s)
    idx_load(0, 0)
    idx_load(1, 1)
    idx_wait(0, 0)
    gather_fire(0)
    plsc.subcore_barrier()

    def outer(jo, carry):
        for par in range(NSLOT):
            j = jo * NSLOT + par
            sl = par
            snx = (par + 1) % NSLOT
            spp = (par + 2) % NSLOT

            @pl.when(j + 2 < nch)
            def _():
                @pl.when(j >= 2)
                def _():
                    scatter_wait(spp)
                idx_load(j + 2, spp)

            @pl.when(j + 1 < nch)
            def _():
                idx_wait(j + 1, snx)
                gather_fire(snx)

            @pl.when(j < nch)
            def _():
                gather_drain(sl)
                compute(sl)
                pltpu.async_copy(rbuf.at[sl], acc.at[rowsc.at[sl]],
                                 ssems[sl], add=True)
        return carry

    lax.fori_loop(0, (nch + NSLOT - 1) // NSLOT, outer, 0)
    for q in range(nch - NSLOT, nch):
        scatter_wait(q % NSLOT)
    plsc.subcore_barrier()
    _drain_acc(acc, out_h, s, c * N)


def _ring_scratch():
    return [
        pltpu.VMEM((NSLOT, CHUNK), jnp.int32),
        pltpu.VMEM((NSLOT, CHUNK), jnp.int32),
        pltpu.VMEM((NSLOT, CHUNK), jnp.int32),
        pltpu.VMEM((NSLOT, CHUNK), jnp.int32),
        pltpu.VMEM((NSLOT, CHUNK), jnp.int32),
        pltpu.VMEM((NSLOT, CHUNK), jnp.int32),
        pltpu.VMEM((NSLOT, CHUNK), jnp.float32),
        pltpu.VMEM((NSLOT, CHUNK), jnp.float32),
        pltpu.VMEM((NSLOT, CHUNK), jnp.float32),
        pltpu.VMEM((NSLOT, CHUNK), jnp.float32),
        pltpu.VMEM((CHUNK,), jnp.float32),
        pltpu.VMEM((CHUNK,), jnp.float32),
        pltpu.VMEM((NSLOT, CHUNK, NFEAT), jnp.float32),
        pltpu.VMEM_SHARED((N, NFEAT), jnp.float32),
    ] + [pltpu.SemaphoreType.DMA] * 9


def _agg1_body(row_h, col_h, tab_h, src_h, srcb_h, z_h, out_h,
               rowi, coli, rowsc, gidx, i0, i2, s0, s1, s2, s3,
               w0, w1, rbuf, acc, *sems):
    c = lax.axis_index("c")
    s = lax.axis_index("s")
    _agg_body_common(True, c, s, row_h, col_h, tab_h, src_h, srcb_h, z_h,
                     out_h, rowi, coli, rowsc, gidx, i0, i2, s0, s1, s2, s3,
                     w0, w1, rbuf, acc, sems[0:3], sems[3:6], sems[6:9])


def _agg1(row, col, tab1, src1, srcb1, zrows):
    mesh = plsc.VectorSubcoreMesh(core_axis_name="c", subcore_axis_name="s")
    f = pl.kernel(
        _agg1_body,
        out_type=jax.ShapeDtypeStruct((2 * N, NFEAT), jnp.float32),
        mesh=mesh,
        scratch_types=_ring_scratch(),
    )
    return f(row, col, tab1, src1, srcb1, zrows)


def _agg2_body(row_h, col_h, tab_h, src_h, z_h, out_h,
               rowi, coli, rowsc, gidx, i0, i2, s0, s1, s2, s3,
               w0, w1, rbuf, acc, *sems):
    c = lax.axis_index("c")
    s = lax.axis_index("s")
    _agg_body_common(False, c, s, row_h, col_h, tab_h, src_h, None, z_h,
                     out_h, rowi, coli, rowsc, gidx, i0, i2, s0, s1, s2, s3,
                     w0, w1, rbuf, acc, sems[0:3], sems[3:6], sems[6:9])


def _agg2(row, col, tab2, src2, zrows):
    mesh = plsc.VectorSubcoreMesh(core_axis_name="c", subcore_axis_name="s")
    f = pl.kernel(
        _agg2_body,
        out_type=jax.ShapeDtypeStruct((2 * N, NFEAT), jnp.float32),
        mesh=mesh,
        scratch_types=_ring_scratch(),
    )
    return f(row, col, tab2, src2, zrows)


def kernel(embeds, edge_index, W, b, a, g1, be1, Wo, bo, ao, go, beo):
    row = edge_index[0]
    col = edge_index[1]

    wt = W.reshape(NHEADS * NHID, NFEAT).T
    bvec = b.reshape(1, NHEADS * NHID)
    p = jnp.zeros((NHEADS * NHID, 8), jnp.float32)
    for h in range(NHEADS):
        p = p.at[h * NHID:(h + 1) * NHID, h].set(a[h, :NHID])
        p = p.at[h * NHID:(h + 1) * NHID, 4 + h].set(a[h, NHID:])

    hp, src = _proj1(embeds, wt, bvec, p)
    tab1 = hp.reshape(2 * N, NFEAT)
    src1 = src.reshape(8 * N)
    srcb1 = jnp.concatenate([src1[1:], jnp.zeros((1,), jnp.float32)])
    zrows = jnp.zeros((ZROWS, NFEAT), jnp.float32)

    hagg = _agg1(row, col, tab1, src1, srcb1, zrows)

    wot = Wo.T
    p2 = jnp.stack([ao[:NFEAT], ao[NFEAT:]], axis=1)
    hout, src2 = _mid(hagg.reshape(2, N, NFEAT), wot, bo.reshape(1, NFEAT),
                      g1.reshape(1, NHEADS * NHID), be1.reshape(1, NHEADS * NHID),
                      p2)

    part = _agg2(row, col, hout, src2.reshape(2 * N), zrows)

    return _fin(part.reshape(2, N, NFEAT), go.reshape(1, NFEAT),
                beo.reshape(1, NFEAT))


# speedup vs baseline: 1.4608x; 1.4608x over previous
import functools

import jax
import jax.numpy as jnp
from jax import lax
from jax.experimental import pallas as pl
from jax.experimental.pallas import tpu as pltpu
from jax.experimental.pallas import tpu_sc as plsc

N = 10000
E = 320000
NFEAT = 128
NHID = 64
NHEADS = 4
ALPHA = 0.2
EPS = 1e-5

NC = 2
NS = 16
CHUNK = 80
ZROWS = 640
ZLAST = N - 15 * ZROWS
BN = 400


def _proj1_body(x_ref, wt_ref, b_ref, p_ref, hp_ref, src_ref):
    acc = jnp.dot(x_ref[...], wt_ref[...],
                  preferred_element_type=jnp.float32) + b_ref[...]
    hp_ref[0] = acc[:, :NFEAT]
    hp_ref[1] = acc[:, NFEAT:]
    src_ref[...] = jnp.dot(acc, p_ref[...], preferred_element_type=jnp.float32)


def _proj1(x, wt, bvec, p):
    return pl.pallas_call(
        _proj1_body,
        grid=(N // BN,),
        in_specs=[
            pl.BlockSpec((BN, NFEAT), lambda i: (i, 0)),
            pl.BlockSpec((NFEAT, 2 * NFEAT), lambda i: (0, 0)),
            pl.BlockSpec((1, 2 * NFEAT), lambda i: (0, 0)),
            pl.BlockSpec((2 * NFEAT, 8), lambda i: (0, 0)),
        ],
        out_specs=[
            pl.BlockSpec((2, BN, NFEAT), lambda i: (0, i, 0)),
            pl.BlockSpec((BN, 8), lambda i: (i, 0)),
        ],
        out_shape=[
            jax.ShapeDtypeStruct((2, N, NFEAT), jnp.float32),
            jax.ShapeDtypeStruct((N, 8), jnp.float32),
        ],
    )(x, wt, bvec, p)


def _ln(x, g, b):
    m = jnp.mean(x, axis=1, keepdims=True)
    v = jnp.mean((x - m) ** 2, axis=1, keepdims=True)
    return (x - m) / jnp.sqrt(v + EPS) * g + b


def _elu(x):
    return jnp.where(x > 0, x, jnp.exp(x) - 1.0)


def _mid_body(hp_ref, wot_ref, bo_ref, g_ref, be_ref, p2_ref, out_ref,
              src2_ref):
    parts = []
    for c in range(2):
        hcat = hp_ref[c]
        for k in range(2):
            h = 2 * c + k
            sl = hcat[:, k * NHID:(k + 1) * NHID]
            g = g_ref[0:1, h * NHID:(h + 1) * NHID]
            be = be_ref[0:1, h * NHID:(h + 1) * NHID]
            parts.append(_elu(_ln(sl, g, be)))
    x = jnp.concatenate(parts, axis=1)
    acc = jnp.dot(x, wot_ref[...], preferred_element_type=jnp.float32) + bo_ref[...]
    out_ref[...] = acc
    src2_ref[...] = jnp.dot(acc, p2_ref[...], preferred_element_type=jnp.float32)


def _mid(hp, wot, bo, g1v, be1v, p2):
    return pl.pallas_call(
        _mid_body,
        grid=(N // BN,),
        in_specs=[
            pl.BlockSpec((2, BN, NFEAT), lambda i: (0, i, 0)),
            pl.BlockSpec((2 * NFEAT, NFEAT), lambda i: (0, 0)),
            pl.BlockSpec((1, NFEAT), lambda i: (0, 0)),
            pl.BlockSpec((1, 2 * NFEAT), lambda i: (0, 0)),
            pl.BlockSpec((1, 2 * NFEAT), lambda i: (0, 0)),
            pl.BlockSpec((NFEAT, 2), lambda i: (0, 0)),
        ],
        out_specs=[
            pl.BlockSpec((BN, NFEAT), lambda i: (i, 0)),
            pl.BlockSpec((BN, 2), lambda i: (i, 0)),
        ],
        out_shape=[
            jax.ShapeDtypeStruct((N, NFEAT), jnp.float32),
            jax.ShapeDtypeStruct((N, 2), jnp.float32),
        ],
    )(hp, wot, bo, g1v, be1v, p2)


def _fin_body(p_ref, go_ref, beo_ref, out_ref):
    sm = p_ref[0] + p_ref[1]
    out_ref[...] = _elu(_ln(sm, go_ref[...], beo_ref[...]))


def _fin(part, gov, beov):
    return pl.pallas_call(
        _fin_body,
        grid=(N // BN,),
        in_specs=[
            pl.BlockSpec((2, BN, NFEAT), lambda i: (0, i, 0)),
            pl.BlockSpec((1, NFEAT), lambda i: (0, 0)),
            pl.BlockSpec((1, NFEAT), lambda i: (0, 0)),
        ],
        out_specs=pl.BlockSpec((BN, NFEAT), lambda i: (i, 0)),
        out_shape=jax.ShapeDtypeStruct((N, NFEAT), jnp.float32),
    )(part, gov, beov)


def _weights16(sa, sb):
    s = sa + sb
    return jnp.exp(-jnp.where(s >= 0, s, ALPHA * s))


def _scale_rows(rbuf, sl, w0, w1, nvec_lo, nvec_hi):
    def group(g, carry):
        gb = pl.multiple_of(g * 16, 16)
        wa_v = w0[pl.ds(gb, 16)]
        wb_v = w1[pl.ds(gb, 16)]
        for l in range(16):
            e = gb + l
            wa = wa_v[l]
            wb = wb_v[l]
            for j in range(nvec_lo):
                d = pl.ds(j * 16, 16)
                rbuf[sl, e, d] = rbuf[sl, e, d] * wa
            for j in range(nvec_lo, nvec_hi):
                d = pl.ds(j * 16, 16)
                rbuf[sl, e, d] = rbuf[sl, e, d] * wb
        return carry
    lax.fori_loop(0, CHUNK // 16, group, 0)


def _zero_acc(z_h, acc, s):
    @pl.when(s < NS - 1)
    def _():
        pltpu.sync_copy(z_h, acc.at[pl.ds(pl.multiple_of(s * ZROWS, 8), ZROWS)])

    @pl.when(s == NS - 1)
    def _():
        pltpu.sync_copy(z_h.at[pl.ds(0, ZLAST)],
                        acc.at[pl.ds((NS - 1) * ZROWS, ZLAST)])


def _drain_acc(acc, out_h, s, base):
    @pl.when(s < NS - 1)
    def _():
        pltpu.sync_copy(acc.at[pl.ds(pl.multiple_of(s * ZROWS, 8), ZROWS)],
                        out_h.at[pl.ds(pl.multiple_of(base + s * ZROWS, 8),
                                       ZROWS)])

    @pl.when(s == NS - 1)
    def _():
        pltpu.sync_copy(acc.at[pl.ds((NS - 1) * ZROWS, ZLAST)],
                        out_h.at[pl.ds(pl.multiple_of(base + (NS - 1) * ZROWS, 8),
                                       ZLAST)])


NCH1 = (E // NS) // CHUNK
NCH2 = (E // (NS * NC)) // CHUNK
NSLOT = 4


def _agg_body_common(layer1, c, s, row_h, col_h, tab_h, src_h, srcb_h, z_h,
                     out_h, rowi, coli, rowsc, gidx, i0, i2, s0, s1, s2, s3,
                     w0, w1, rbuf, acc, isems, gsems, ssems):
    if layer1:
        epw = E // NS
        nch = NCH1
        base = s * epw
        h0 = 2 * c
    else:
        epw = E // (NS * NC)
        nch = NCH2
        base = (s * NC + c) * epw

    def idx_load(q, sl):
        b = pl.multiple_of(base + q * CHUNK, 16)
        pltpu.async_copy(row_h.at[pl.ds(b, CHUNK)], rowi.at[sl], isems[sl])
        pltpu.async_copy(col_h.at[pl.ds(b, CHUNK)], coli.at[sl], isems[sl])

    def idx_wait(q, sl):
        b = pl.multiple_of(base + q * CHUNK, 16)
        pltpu.make_async_copy(row_h.at[pl.ds(b, CHUNK)], rowi.at[sl],
                              isems[sl]).wait()
        pltpu.make_async_copy(col_h.at[pl.ds(b, CHUNK)], coli.at[sl],
                              isems[sl]).wait()

    def gather_fire(sl):
        for k in range(CHUNK // 16):
            d = pl.ds(k * 16, 16)
            r = rowi[sl, d]
            cc = coli[sl, d]
            if layer1:
                gidx[sl, d] = cc + c * N
                i0[sl, d] = r * 8 + h0
                i2[sl, d] = cc * 8 + (h0 + 4)
            else:
                gidx[sl, d] = cc
                i0[sl, d] = r * 2
                i2[sl, d] = cc * 2 + 1
        pltpu.async_copy(tab_h.at[gidx.at[sl]], rbuf.at[sl], gsems[sl])
        pltpu.async_copy(src_h.at[i0.at[sl]], s0.at[sl], gsems[sl])
        pltpu.async_copy(src_h.at[i2.at[sl]], s2.at[sl], gsems[sl])
        if layer1:
            pltpu.async_copy(srcb_h.at[i0.at[sl]], s1.at[sl], gsems[sl])
            pltpu.async_copy(srcb_h.at[i2.at[sl]], s3.at[sl], gsems[sl])

    def gather_drain(sl):
        pltpu.make_async_copy(tab_h.at[gidx.at[sl]], rbuf.at[sl],
                              gsems[sl]).wait()
        pltpu.make_async_copy(src_h.at[i0.at[sl]], s0.at[sl], gsems[sl]).wait()
        pltpu.make_async_copy(src_h.at[i2.at[sl]], s2.at[sl], gsems[sl]).wait()
        if layer1:
            pltpu.make_async_copy(srcb_h.at[i0.at[sl]], s1.at[sl],
                                  gsems[sl]).wait()
            pltpu.make_async_copy(srcb_h.at[i2.at[sl]], s3.at[sl],
                                  gsems[sl]).wait()

    def scatter_wait(sl):
        pltpu.make_async_copy(rbuf.at[sl], acc.at[rowsc.at[sl]],
                              ssems[sl]).wait()

    def compute(sl):
        for k in range(CHUNK // 16):
            d = pl.ds(k * 16, 16)
            rowsc[sl, d] = rowi[sl, d]
            if layer1:
                w0[d] = _weights16(s0[sl, d], s2[sl, d])
                w1[d] = _weights16(s1[sl, d], s3[sl, d])
            else:
                w0[d] = _weights16(s0[sl, d], s2[sl, d])
        if layer1:
            _scale_rows(rbuf, sl, w0, w1, 4, 8)
        else:
            _scale_rows(rbuf, sl, w0, w0, 8, 8)

    _zero_acc(z_h, acc, s)
    idx_load(0, 0)
    idx_load(1, 1)
    idx_load(2, 2)
    idx_wait(0, 0)
    gather_fire(0)
    idx_wait(1, 1)
    gather_fire(1)
    plsc.subcore_barrier()

    def outer(jo, carry):
        for par in range(NSLOT):
            j = jo * NSLOT + par
            sl = par
            spp2 = (par + 2) % NSLOT
            spp3 = (par + 3) % NSLOT

            @pl.when(j + 3 < nch)
            def _():
                idx_load(j + 3, spp3)

            @pl.when(j + 2 < nch)
            def _():
                @pl.when(j >= 2)
                def _():
                    scatter_wait(spp2)
                idx_wait(j + 2, spp2)
                gather_fire(spp2)

            @pl.when(j < nch)
            def _():
                gather_drain(sl)
                compute(sl)
                pltpu.async_copy(rbuf.at[sl], acc.at[rowsc.at[sl]],
                                 ssems[sl], add=True)
        return carry

    lax.fori_loop(0, (nch + NSLOT - 1) // NSLOT, outer, 0)
    for q in range(nch - NSLOT, nch):
        scatter_wait(q % NSLOT)
    plsc.subcore_barrier()
    _drain_acc(acc, out_h, s, c * N)


def _ring_scratch():
    return [
        pltpu.VMEM((NSLOT, CHUNK), jnp.int32),
        pltpu.VMEM((NSLOT, CHUNK), jnp.int32),
        pltpu.VMEM((NSLOT, CHUNK), jnp.int32),
        pltpu.VMEM((NSLOT, CHUNK), jnp.int32),
        pltpu.VMEM((NSLOT, CHUNK), jnp.int32),
        pltpu.VMEM((NSLOT, CHUNK), jnp.int32),
        pltpu.VMEM((NSLOT, CHUNK), jnp.float32),
        pltpu.VMEM((NSLOT, CHUNK), jnp.float32),
        pltpu.VMEM((NSLOT, CHUNK), jnp.float32),
        pltpu.VMEM((NSLOT, CHUNK), jnp.float32),
        pltpu.VMEM((CHUNK,), jnp.float32),
        pltpu.VMEM((CHUNK,), jnp.float32),
        pltpu.VMEM((NSLOT, CHUNK, NFEAT), jnp.float32),
        pltpu.VMEM_SHARED((N, NFEAT), jnp.float32),
    ] + [pltpu.SemaphoreType.DMA] * (3 * NSLOT)


def _agg1_body(row_h, col_h, tab_h, src_h, srcb_h, z_h, out_h,
               rowi, coli, rowsc, gidx, i0, i2, s0, s1, s2, s3,
               w0, w1, rbuf, acc, *sems):
    c = lax.axis_index("c")
    s = lax.axis_index("s")
    _agg_body_common(True, c, s, row_h, col_h, tab_h, src_h, srcb_h, z_h,
                     out_h, rowi, coli, rowsc, gidx, i0, i2, s0, s1, s2, s3,
                     w0, w1, rbuf, acc, sems[0:NSLOT], sems[NSLOT:2 * NSLOT], sems[2 * NSLOT:3 * NSLOT])


def _agg1(row, col, tab1, src1, srcb1, zrows):
    mesh = plsc.VectorSubcoreMesh(core_axis_name="c", subcore_axis_name="s")
    f = pl.kernel(
        _agg1_body,
        out_type=jax.ShapeDtypeStruct((2 * N, NFEAT), jnp.float32),
        mesh=mesh,
        scratch_types=_ring_scratch(),
    )
    return f(row, col, tab1, src1, srcb1, zrows)


def _agg2_body(row_h, col_h, tab_h, src_h, z_h, out_h,
               rowi, coli, rowsc, gidx, i0, i2, s0, s1, s2, s3,
               w0, w1, rbuf, acc, *sems):
    c = lax.axis_index("c")
    s = lax.axis_index("s")
    _agg_body_common(False, c, s, row_h, col_h, tab_h, src_h, None, z_h,
                     out_h, rowi, coli, rowsc, gidx, i0, i2, s0, s1, s2, s3,
                     w0, w1, rbuf, acc, sems[0:NSLOT], sems[NSLOT:2 * NSLOT], sems[2 * NSLOT:3 * NSLOT])


def _agg2(row, col, tab2, src2, zrows):
    mesh = plsc.VectorSubcoreMesh(core_axis_name="c", subcore_axis_name="s")
    f = pl.kernel(
        _agg2_body,
        out_type=jax.ShapeDtypeStruct((2 * N, NFEAT), jnp.float32),
        mesh=mesh,
        scratch_types=_ring_scratch(),
    )
    return f(row, col, tab2, src2, zrows)


def kernel(embeds, edge_index, W, b, a, g1, be1, Wo, bo, ao, go, beo):
    row = edge_index[0]
    col = edge_index[1]

    wt = W.reshape(NHEADS * NHID, NFEAT).T
    bvec = b.reshape(1, NHEADS * NHID)
    p = jnp.zeros((NHEADS * NHID, 8), jnp.float32)
    for h in range(NHEADS):
        p = p.at[h * NHID:(h + 1) * NHID, h].set(a[h, :NHID])
        p = p.at[h * NHID:(h + 1) * NHID, 4 + h].set(a[h, NHID:])

    hp, src = _proj1(embeds, wt, bvec, p)
    tab1 = hp.reshape(2 * N, NFEAT)
    src1 = src.reshape(8 * N)
    srcb1 = jnp.concatenate([src1[1:], jnp.zeros((1,), jnp.float32)])
    zrows = jnp.zeros((ZROWS, NFEAT), jnp.float32)

    hagg = _agg1(row, col, tab1, src1, srcb1, zrows)

    wot = Wo.T
    p2 = jnp.stack([ao[:NFEAT], ao[NFEAT:]], axis=1)
    hout, src2 = _mid(hagg.reshape(2, N, NFEAT), wot, bo.reshape(1, NFEAT),
                      g1.reshape(1, NHEADS * NHID), be1.reshape(1, NHEADS * NHID),
                      p2)

    part = _agg2(row, col, hout, src2.reshape(2 * N), zrows)

    return _fin(part.reshape(2, N, NFEAT), go.reshape(1, NFEAT),
                beo.reshape(1, NFEAT))


# speedup vs baseline: 1.4894x; 1.0196x over previous
import functools

import jax
import jax.numpy as jnp
from jax import lax
from jax.experimental import pallas as pl
from jax.experimental.pallas import tpu as pltpu
from jax.experimental.pallas import tpu_sc as plsc

N = 10000
E = 320000
NFEAT = 128
NHID = 64
NHEADS = 4
ALPHA = 0.2
EPS = 1e-5

NC = 2
NS = 16
CHUNK = 80
ZROWS = 640
ZLAST = N - 15 * ZROWS
BN = 400


def _proj1_body(x_ref, wt_ref, b_ref, hp_ref, src_ref):
    acc = jnp.dot(x_ref[...], wt_ref[...],
                  preferred_element_type=jnp.float32) + b_ref[...]
    hp_ref[0] = acc[:, :NFEAT]
    hp_ref[1] = acc[:, NFEAT:2 * NFEAT]
    src_ref[...] = acc[:, 2 * NFEAT:]


def _proj1(x, wtp, bfull):
    return pl.pallas_call(
        _proj1_body,
        grid=(N // BN,),
        in_specs=[
            pl.BlockSpec((BN, NFEAT), lambda i: (i, 0)),
            pl.BlockSpec((NFEAT, 2 * NFEAT + 8), lambda i: (0, 0)),
            pl.BlockSpec((1, 2 * NFEAT + 8), lambda i: (0, 0)),
        ],
        out_specs=[
            pl.BlockSpec((2, BN, NFEAT), lambda i: (0, i, 0)),
            pl.BlockSpec((BN, 8), lambda i: (i, 0)),
        ],
        out_shape=[
            jax.ShapeDtypeStruct((2, N, NFEAT), jnp.float32),
            jax.ShapeDtypeStruct((N, 8), jnp.float32),
        ],
    )(x, wtp, bfull)


def _ln(x, g, b):
    m = jnp.mean(x, axis=1, keepdims=True)
    v = jnp.mean((x - m) ** 2, axis=1, keepdims=True)
    return (x - m) / jnp.sqrt(v + EPS) * g + b


def _elu(x):
    return jnp.where(x > 0, x, jnp.exp(x) - 1.0)


def _mid_body(hp_ref, wot_ref, bo_ref, g_ref, be_ref, out_ref, src2_ref):
    parts = []
    for c in range(2):
        hcat = hp_ref[c]
        for k in range(2):
            h = 2 * c + k
            sl = hcat[:, k * NHID:(k + 1) * NHID]
            g = g_ref[0:1, h * NHID:(h + 1) * NHID]
            be = be_ref[0:1, h * NHID:(h + 1) * NHID]
            parts.append(_elu(_ln(sl, g, be)))
    x = jnp.concatenate(parts, axis=1)
    acc = jnp.dot(x, wot_ref[...], preferred_element_type=jnp.float32) + bo_ref[...]
    out_ref[...] = acc[:, :NFEAT]
    src2_ref[...] = acc[:, NFEAT:]


def _mid(hp, wot_all, bo_all, g1v, be1v):
    return pl.pallas_call(
        _mid_body,
        grid=(N // BN,),
        in_specs=[
            pl.BlockSpec((2, BN, NFEAT), lambda i: (0, i, 0)),
            pl.BlockSpec((2 * NFEAT, NFEAT + 2), lambda i: (0, 0)),
            pl.BlockSpec((1, NFEAT + 2), lambda i: (0, 0)),
            pl.BlockSpec((1, 2 * NFEAT), lambda i: (0, 0)),
            pl.BlockSpec((1, 2 * NFEAT), lambda i: (0, 0)),
        ],
        out_specs=[
            pl.BlockSpec((BN, NFEAT), lambda i: (i, 0)),
            pl.BlockSpec((BN, 2), lambda i: (i, 0)),
        ],
        out_shape=[
            jax.ShapeDtypeStruct((N, NFEAT), jnp.float32),
            jax.ShapeDtypeStruct((N, 2), jnp.float32),
        ],
    )(hp, wot_all, bo_all, g1v, be1v)


def _fin_body(p_ref, go_ref, beo_ref, out_ref):
    sm = p_ref[0] + p_ref[1]
    out_ref[...] = _elu(_ln(sm, go_ref[...], beo_ref[...]))


def _fin(part, gov, beov):
    return pl.pallas_call(
        _fin_body,
        grid=(N // BN,),
        in_specs=[
            pl.BlockSpec((2, BN, NFEAT), lambda i: (0, i, 0)),
            pl.BlockSpec((1, NFEAT), lambda i: (0, 0)),
            pl.BlockSpec((1, NFEAT), lambda i: (0, 0)),
        ],
        out_specs=pl.BlockSpec((BN, NFEAT), lambda i: (i, 0)),
        out_shape=jax.ShapeDtypeStruct((N, NFEAT), jnp.float32),
    )(part, gov, beov)


def _weights16(sa, sb):
    s = sa + sb
    return jnp.exp(-jnp.where(s >= 0, s, ALPHA * s))


def _scale_rows(rbuf, sl, w0, w1, nvec_lo, nvec_hi):
    def group(g, carry):
        gb = pl.multiple_of(g * 16, 16)
        wa_v = w0[pl.ds(gb, 16)]
        wb_v = w1[pl.ds(gb, 16)]
        for l in range(16):
            e = gb + l
            wa = wa_v[l]
            wb = wb_v[l]
            for j in range(nvec_lo):
                d = pl.ds(j * 16, 16)
                rbuf[sl, e, d] = rbuf[sl, e, d] * wa
            for j in range(nvec_lo, nvec_hi):
                d = pl.ds(j * 16, 16)
                rbuf[sl, e, d] = rbuf[sl, e, d] * wb
        return carry
    lax.fori_loop(0, CHUNK // 16, group, 0)


def _zero_acc(z_h, acc, s):
    @pl.when(s < NS - 1)
    def _():
        pltpu.sync_copy(z_h, acc.at[pl.ds(pl.multiple_of(s * ZROWS, 8), ZROWS)])

    @pl.when(s == NS - 1)
    def _():
        pltpu.sync_copy(z_h.at[pl.ds(0, ZLAST)],
                        acc.at[pl.ds((NS - 1) * ZROWS, ZLAST)])


def _drain_acc(acc, out_h, s, base):
    @pl.when(s < NS - 1)
    def _():
        pltpu.sync_copy(acc.at[pl.ds(pl.multiple_of(s * ZROWS, 8), ZROWS)],
                        out_h.at[pl.ds(pl.multiple_of(base + s * ZROWS, 8),
                                       ZROWS)])

    @pl.when(s == NS - 1)
    def _():
        pltpu.sync_copy(acc.at[pl.ds((NS - 1) * ZROWS, ZLAST)],
                        out_h.at[pl.ds(pl.multiple_of(base + (NS - 1) * ZROWS, 8),
                                       ZLAST)])


NCH1 = (E // NS) // CHUNK
NCH2 = (E // (NS * NC)) // CHUNK
NSLOT = 4


def _agg_body_common(layer1, c, s, row_h, col_h, tab_h, src_h, srcb_h, z_h,
                     out_h, rowi, coli, rowsc, gidx, i0, i2, s0, s1, s2, s3,
                     w0, w1, rbuf, acc, isems, gsems, ssems):
    if layer1:
        epw = E // NS
        nch = NCH1
        base = s * epw
        h0 = 2 * c
    else:
        epw = E // (NS * NC)
        nch = NCH2
        base = (s * NC + c) * epw

    def idx_load(q, sl):
        b = pl.multiple_of(base + q * CHUNK, 16)
        pltpu.async_copy(row_h.at[pl.ds(b, CHUNK)], rowi.at[sl], isems[sl])
        pltpu.async_copy(col_h.at[pl.ds(b, CHUNK)], coli.at[sl], isems[sl])

    def idx_wait(q, sl):
        b = pl.multiple_of(base + q * CHUNK, 16)
        pltpu.make_async_copy(row_h.at[pl.ds(b, CHUNK)], rowi.at[sl],
                              isems[sl]).wait()
        pltpu.make_async_copy(col_h.at[pl.ds(b, CHUNK)], coli.at[sl],
                              isems[sl]).wait()

    def gather_fire(sl):
        if layer1:
            for k in range(CHUNK // 16):
                d = pl.ds(k * 16, 16)
                gidx[sl, d] = coli[sl, d] + c * N
                i0[sl, d] = rowi[sl, d] * 8 + h0
                i2[sl, d] = coli[sl, d] * 8 + (h0 + 4)
            pltpu.async_copy(tab_h.at[gidx.at[sl]], rbuf.at[sl], gsems[sl])
            pltpu.async_copy(src_h.at[i0.at[sl]], s0.at[sl], gsems[sl])
            pltpu.async_copy(srcb_h.at[i0.at[sl]], s1.at[sl], gsems[sl])
            pltpu.async_copy(src_h.at[i2.at[sl]], s2.at[sl], gsems[sl])
            pltpu.async_copy(srcb_h.at[i2.at[sl]], s3.at[sl], gsems[sl])
        else:
            pltpu.async_copy(tab_h.at[coli.at[sl]], rbuf.at[sl], gsems[sl])
            pltpu.async_copy(src_h.at[rowi.at[sl]], s0.at[sl], gsems[sl])
            pltpu.async_copy(srcb_h.at[coli.at[sl]], s2.at[sl], gsems[sl])

    def gather_drain(sl):
        if layer1:
            pltpu.make_async_copy(tab_h.at[gidx.at[sl]], rbuf.at[sl],
                                  gsems[sl]).wait()
            pltpu.make_async_copy(src_h.at[i0.at[sl]], s0.at[sl],
                                  gsems[sl]).wait()
            pltpu.make_async_copy(srcb_h.at[i0.at[sl]], s1.at[sl],
                                  gsems[sl]).wait()
            pltpu.make_async_copy(src_h.at[i2.at[sl]], s2.at[sl],
                                  gsems[sl]).wait()
            pltpu.make_async_copy(srcb_h.at[i2.at[sl]], s3.at[sl],
                                  gsems[sl]).wait()
        else:
            pltpu.make_async_copy(tab_h.at[coli.at[sl]], rbuf.at[sl],
                                  gsems[sl]).wait()
            pltpu.make_async_copy(src_h.at[rowi.at[sl]], s0.at[sl],
                                  gsems[sl]).wait()
            pltpu.make_async_copy(srcb_h.at[coli.at[sl]], s2.at[sl],
                                  gsems[sl]).wait()

    def scatter_wait(sl):
        pltpu.make_async_copy(rbuf.at[sl], acc.at[rowsc.at[sl]],
                              ssems[sl]).wait()

    def compute(sl):
        for k in range(CHUNK // 16):
            d = pl.ds(k * 16, 16)
            rowsc[sl, d] = rowi[sl, d]
            if layer1:
                w0[d] = _weights16(s0[sl, d], s2[sl, d])
                w1[d] = _weights16(s1[sl, d], s3[sl, d])
            else:
                w0[d] = _weights16(s0[sl, d], s2[sl, d])
        if layer1:
            _scale_rows(rbuf, sl, w0, w1, 4, 8)
        else:
            _scale_rows(rbuf, sl, w0, w0, 8, 8)

    _zero_acc(z_h, acc, s)
    idx_load(0, 0)
    idx_load(1, 1)
    idx_load(2, 2)
    idx_wait(0, 0)
    gather_fire(0)
    idx_wait(1, 1)
    gather_fire(1)
    plsc.subcore_barrier()

    def outer(jo, carry):
        for par in range(NSLOT):
            j = jo * NSLOT + par
            sl = par
            spp2 = (par + 2) % NSLOT
            spp3 = (par + 3) % NSLOT

            @pl.when(j + 3 < nch)
            def _():
                idx_load(j + 3, spp3)

            @pl.when(j + 2 < nch)
            def _():
                @pl.when(j >= 2)
                def _():
                    scatter_wait(spp2)
                idx_wait(j + 2, spp2)
                gather_fire(spp2)

            @pl.when(j < nch)
            def _():
                gather_drain(sl)
                compute(sl)
                pltpu.async_copy(rbuf.at[sl], acc.at[rowsc.at[sl]],
                                 ssems[sl], add=True)
        return carry

    lax.fori_loop(0, (nch + NSLOT - 1) // NSLOT, outer, 0)
    for q in range(nch - NSLOT, nch):
        scatter_wait(q % NSLOT)
    plsc.subcore_barrier()
    _drain_acc(acc, out_h, s, c * N)


def _ring_scratch(layer1):
    base = [
        pltpu.VMEM((NSLOT, CHUNK), jnp.int32),
        pltpu.VMEM((NSLOT, CHUNK), jnp.int32),
        pltpu.VMEM((NSLOT, CHUNK), jnp.int32),
    ]
    if layer1:
        base += [
            pltpu.VMEM((NSLOT, CHUNK), jnp.int32),
            pltpu.VMEM((NSLOT, CHUNK), jnp.int32),
            pltpu.VMEM((NSLOT, CHUNK), jnp.int32),
            pltpu.VMEM((NSLOT, CHUNK), jnp.float32),
            pltpu.VMEM((NSLOT, CHUNK), jnp.float32),
            pltpu.VMEM((NSLOT, CHUNK), jnp.float32),
            pltpu.VMEM((NSLOT, CHUNK), jnp.float32),
        ]
    else:
        base += [
            pltpu.VMEM((NSLOT, CHUNK), jnp.float32),
            pltpu.VMEM((NSLOT, CHUNK), jnp.float32),
        ]
    return base + [
        pltpu.VMEM((CHUNK,), jnp.float32),
        pltpu.VMEM((CHUNK,), jnp.float32),
        pltpu.VMEM((NSLOT, CHUNK, NFEAT), jnp.float32),
        pltpu.VMEM_SHARED((N, NFEAT), jnp.float32),
    ] + [pltpu.SemaphoreType.DMA] * (3 * NSLOT)


def _agg1_body(row_h, col_h, tab_h, src_h, srcb_h, z_h, out_h,
               rowi, coli, rowsc, gidx, i0, i2, s0, s1, s2, s3,
               w0, w1, rbuf, acc, *sems):
    c = lax.axis_index("c")
    s = lax.axis_index("s")
    _agg_body_common(True, c, s, row_h, col_h, tab_h, src_h, srcb_h, z_h,
                     out_h, rowi, coli, rowsc, gidx, i0, i2, s0, s1, s2, s3,
                     w0, w1, rbuf, acc, sems[0:NSLOT], sems[NSLOT:2 * NSLOT],
                     sems[2 * NSLOT:3 * NSLOT])


def _agg1(row, col, tab1, src1, srcb1, zrows):
    mesh = plsc.VectorSubcoreMesh(core_axis_name="c", subcore_axis_name="s")
    f = pl.kernel(
        _agg1_body,
        out_type=jax.ShapeDtypeStruct((2 * N, NFEAT), jnp.float32),
        mesh=mesh,
        scratch_types=_ring_scratch(True),
    )
    return f(row, col, tab1, src1, srcb1, zrows)


def _agg2_body(row_h, col_h, tab_h, sr_h, sc_h, z_h, out_h,
               rowi, coli, rowsc, s0, s2,
               w0, w1, rbuf, acc, *sems):
    c = lax.axis_index("c")
    s = lax.axis_index("s")
    _agg_body_common(False, c, s, row_h, col_h, tab_h, sr_h, sc_h, z_h,
                     out_h, rowi, coli, rowsc, None, None, None,
                     s0, None, s2, None,
                     w0, w1, rbuf, acc, sems[0:NSLOT], sems[NSLOT:2 * NSLOT],
                     sems[2 * NSLOT:3 * NSLOT])


def _agg2(row, col, tab2, sr2, sc2, zrows):
    mesh = plsc.VectorSubcoreMesh(core_axis_name="c", subcore_axis_name="s")
    f = pl.kernel(
        _agg2_body,
        out_type=jax.ShapeDtypeStruct((2 * N, NFEAT), jnp.float32),
        mesh=mesh,
        scratch_types=_ring_scratch(False),
    )
    return f(row, col, tab2, sr2, sc2, zrows)


def kernel(embeds, edge_index, W, b, a, g1, be1, Wo, bo, ao, go, beo):
    row = edge_index[0]
    col = edge_index[1]

    wt = W.reshape(NHEADS * NHID, NFEAT).T
    bvec = b.reshape(1, NHEADS * NHID)
    p = jnp.zeros((NHEADS * NHID, 8), jnp.float32)
    for h in range(NHEADS):
        p = p.at[h * NHID:(h + 1) * NHID, h].set(a[h, :NHID])
        p = p.at[h * NHID:(h + 1) * NHID, 4 + h].set(a[h, NHID:])
    wtp = jnp.concatenate([wt, wt @ p], axis=1)
    bfull = jnp.concatenate([bvec, bvec @ p], axis=1)

    hp, src = _proj1(embeds, wtp, bfull)
    tab1 = hp.reshape(2 * N, NFEAT)
    src1 = src.reshape(8 * N)
    srcb1 = jnp.concatenate([src1[1:], jnp.zeros((1,), jnp.float32)])
    zrows = jnp.zeros((ZROWS, NFEAT), jnp.float32)

    hagg = _agg1(row, col, tab1, src1, srcb1, zrows)

    wot = Wo.T
    p2 = jnp.stack([ao[:NFEAT], ao[NFEAT:]], axis=1)
    wot_all = jnp.concatenate([wot, wot @ p2], axis=1)
    bo_all = jnp.concatenate([bo.reshape(1, NFEAT),
                              bo.reshape(1, NFEAT) @ p2], axis=1)
    hout, src2 = _mid(hagg.reshape(2, N, NFEAT), wot_all, bo_all,
                      g1.reshape(1, NHEADS * NHID),
                      be1.reshape(1, NHEADS * NHID))

    sr2 = src2[:, 0]
    sc2 = src2[:, 1]
    part = _agg2(row, col, hout, sr2, sc2, zrows)

    return _fin(part.reshape(2, N, NFEAT), go.reshape(1, NFEAT),
                beo.reshape(1, NFEAT))
